# baseline (speedup 1.0000x reference)
import functools

import jax
import jax.numpy as jnp
from jax.experimental import pallas as pl
from jax.experimental.pallas import tpu as pltpu


def _fused_ln_kernel(x_ref, pgb_ref, o_ref, *, eps, seq_len):
    pos = pgb_ref[:seq_len, :]
    gamma = pgb_ref[seq_len, :]
    beta = pgb_ref[seq_len + 1, :]
    x = x_ref[...] + pos
    m = jnp.mean(x, axis=-1, keepdims=True)
    m2 = jnp.mean(x * x, axis=-1, keepdims=True)
    var = jnp.maximum(m2 - m * m, 0.0)
    inv = jax.lax.rsqrt(var + jnp.float32(eps))
    o_ref[...] = (x - m) * (inv * gamma) + beta


def kernel(input_vis_feats, pos_table, gamma, beta, eps=1e-12):
    B, S, H = input_vis_feats.shape
    pgb = jnp.concatenate(
        [pos_table[:S], gamma.reshape(1, H), beta.reshape(1, H)], axis=0
    )

    itemsize = jnp.dtype(input_vis_feats.dtype).itemsize
    row_bytes = S * H * itemsize
    tb = 1
    while tb < B and B % (tb * 2) == 0 and (tb * 2) * row_bytes <= (12 << 20):
        tb *= 2

    grid = (B // tb,)
    x_spec = pl.BlockSpec((tb, S, H), lambda i: (i, 0, 0))
    return pl.pallas_call(
        functools.partial(_fused_ln_kernel, eps=eps, seq_len=S),
        out_shape=jax.ShapeDtypeStruct((B, S, H), input_vis_feats.dtype),
        grid=grid,
        in_specs=[
            x_spec,
            pl.BlockSpec((S + 2, H), lambda i: (0, 0)),
        ],
        out_specs=x_spec,
        compiler_params=pltpu.CompilerParams(
            dimension_semantics=("parallel",),
            allow_input_fusion=[False, True],
            vmem_limit_bytes=63 << 20,
        ),
    )(input_vis_feats, pgb)


# speedup vs baseline: 1.0500x; 1.0500x over previous
import functools

import jax
import jax.numpy as jnp
from jax.experimental import pallas as pl
from jax.experimental.pallas import tpu as pltpu


def _fused_ln_kernel(x_ref, pgb_ref, o_ref, *, eps, seq_len):
    pos = pgb_ref[:seq_len, :]
    gamma = pgb_ref[seq_len, :]
    beta = pgb_ref[seq_len + 1, :]
    x = x_ref[...] + pos
    m = jnp.mean(x, axis=-1, keepdims=True)
    m2 = jnp.mean(x * x, axis=-1, keepdims=True)
    var = jnp.maximum(m2 - m * m, 0.0)
    inv = jax.lax.rsqrt(var + jnp.float32(eps))
    o_ref[...] = (x - m) * (inv * gamma) + beta


def kernel(input_vis_feats, pos_table, gamma, beta, eps=1e-12):
    B, S, H = input_vis_feats.shape
    pgb = jnp.concatenate(
        [pos_table[:S], gamma.reshape(1, H), beta.reshape(1, H)], axis=0
    )

    itemsize = jnp.dtype(input_vis_feats.dtype).itemsize
    row_bytes = S * H * itemsize
    tb = 1
    while tb < B and B % (tb * 2) == 0 and (tb * 2) * row_bytes <= (6 << 20):
        tb *= 2

    grid = (B // tb,)
    x_spec = pl.BlockSpec((tb, S, H), lambda i: (i, 0, 0))
    return pl.pallas_call(
        functools.partial(_fused_ln_kernel, eps=eps, seq_len=S),
        out_shape=jax.ShapeDtypeStruct((B, S, H), input_vis_feats.dtype),
        grid=grid,
        in_specs=[
            x_spec,
            pl.BlockSpec((S + 2, H), lambda i: (0, 0)),
        ],
        out_specs=x_spec,
        compiler_params=pltpu.CompilerParams(
            dimension_semantics=("parallel",),
            allow_input_fusion=[False, True],
            vmem_limit_bytes=48 << 20,
        ),
    )(input_vis_feats, pgb)
